# baseline (speedup 1.0000x reference)
import functools

import jax
import jax.numpy as jnp
from jax import lax
from jax.experimental import pallas as pl
from jax.experimental.pallas import tpu as pltpu
from jax.experimental.pallas import tpu_sc as plsc

NC = 2
NS = 16
CH = 128
M = 654592


def _make_sc_kernel(B, S, D, V):
    d_per_core = D // NC
    b_per_tile = B // NS
    half = b_per_tile // 2
    n_ch_half = (S * half) // CH
    kh = half // CH
    t_main = (V - V % 128) - M
    t_size = V - M
    arena = 2 * M + t_size + 64
    seg = (M // NS) // 128 * 128
    seg_last = M - seg * (NS - 1)
    tseg = (t_main // NS) // 128 * 128
    tseg_last = t_main - tseg * (NS - 1)
    mesh = plsc.VectorSubcoreMesh(core_axis_name="c", subcore_axis_name="s")

    @functools.partial(
        pl.kernel,
        out_type=jax.ShapeDtypeStruct((D, B), jnp.float32),
        mesh=mesh,
        scratch_types=[
            pltpu.VMEM((S, b_per_tile), jnp.int32),
            pltpu.VMEM((S, b_per_tile), jnp.int32),
            pltpu.VMEM((S, b_per_tile), jnp.float32),
            pltpu.VMEM((b_per_tile,), jnp.float32),
            pltpu.VMEM((D, 128), jnp.float32),
            pltpu.VMEM_SHARED((arena,), jnp.float32),
            pltpu.SemaphoreType.DMA,
            pltpu.SemaphoreType.DMA,
        ],
    )
    def run(tbl_hbm, idx_hbm, aux_hbm, out_hbm, raw_v, ia_v, g_v,
            out_v, aux_v, arena_sh, sem, sem_stage):
        cid = lax.axis_index("c")
        sid = lax.axis_index("s")
        b0 = sid * b_per_tile
        d_base = cid * d_per_core
        c0 = pl.multiple_of(sid * seg, 128)
        tc0 = pl.multiple_of(sid * tseg, 128)

        pltpu.sync_copy(idx_hbm.at[:, pl.ds(b0, b_per_tile)], raw_v)

        @pl.when(sid == NS - 1)
        def _stage_aux():
            pltpu.sync_copy(aux_hbm, aux_v)

        m_c = jnp.int32(M)

        def xform(j, c2):
            s = j // (b_per_tile // 16)
            i = j % (b_per_tile // 16)
            x = raw_v[s, pl.ds(i * 16, 16)]
            ia_v[s, pl.ds(i * 16, 16)] = jnp.where(x >= m_c, x + m_c, x)
            return c2

        lax.fori_loop(0, S * (b_per_tile // 16), xform, 0)

        inv = jnp.float32(1.0 / S)
        ref_b = arena_sh.at[pl.ds(M, V + 64)]

        def stage_main(d, p):
            base = p * M

            @pl.when(sid < NS - 1)
            def _seg():
                pltpu.async_copy(
                    tbl_hbm.at[d].at[pl.ds(c0, seg)],
                    arena_sh.at[pl.ds(base + c0, seg)],
                    sem_stage,
                )

            @pl.when(sid == NS - 1)
            def _seg_last():
                pltpu.async_copy(
                    tbl_hbm.at[d].at[pl.ds(c0, seg_last)],
                    arena_sh.at[pl.ds(base + c0, seg_last)],
                    sem_stage,
                )

        def stage_main_wait():
            @pl.when(sid < NS - 1)
            def _seg():
                pltpu.make_async_copy(
                    tbl_hbm.at[0].at[pl.ds(0, seg)],
                    arena_sh.at[pl.ds(0, seg)],
                    sem_stage,
                ).wait()

            @pl.when(sid == NS - 1)
            def _seg_last():
                pltpu.make_async_copy(
                    tbl_hbm.at[0].at[pl.ds(0, seg_last)],
                    arena_sh.at[pl.ds(0, seg_last)],
                    sem_stage,
                ).wait()

        def stage_tail(d):
            @pl.when(sid < NS - 1)
            def _t():
                pltpu.async_copy(
                    tbl_hbm.at[d].at[pl.ds(M + tc0, tseg)],
                    arena_sh.at[pl.ds(2 * M + tc0, tseg)],
                    sem_stage,
                )

            @pl.when(sid == NS - 1)
            def _t_last():
                pltpu.async_copy(
                    tbl_hbm.at[d].at[pl.ds(M + tc0, tseg_last)],
                    arena_sh.at[pl.ds(2 * M + tc0, tseg_last)],
                    sem_stage,
                )
                pltpu.async_copy(
                    aux_v.at[d],
                    arena_sh.at[pl.ds(2 * M + t_size - 128, 128)],
                    sem_stage,
                )

        def stage_tail_wait():
            @pl.when(sid < NS - 1)
            def _t():
                pltpu.make_async_copy(
                    tbl_hbm.at[0].at[pl.ds(0, tseg)],
                    arena_sh.at[pl.ds(0, tseg)],
                    sem_stage,
                ).wait()

            @pl.when(sid == NS - 1)
            def _t_last():
                pltpu.make_async_copy(
                    tbl_hbm.at[0].at[pl.ds(0, tseg_last)],
                    arena_sh.at[pl.ds(0, tseg_last)],
                    sem_stage,
                ).wait()
                pltpu.make_async_copy(
                    aux_v.at[0],
                    arena_sh.at[pl.ds(0, 128)],
                    sem_stage,
                ).wait()

        def fire(p, h):
            def body(k, c2):
                off = h * half + k * CH
                for s in range(S):
                    if p == 0:
                        src = arena_sh.at[ia_v.at[s, pl.ds(off, CH)]]
                    else:
                        src = ref_b.at[raw_v.at[s, pl.ds(off, CH)]]
                    pltpu.async_copy(src, g_v.at[s, pl.ds(off, CH)], sem)
                return c2

            lax.fori_loop(0, kh, body, 0)

        def drain(h):
            pltpu.make_async_copy(
                tbl_hbm.at[pl.ds(0, S), pl.ds(0, half)],
                g_v.at[:, pl.ds(h * half, half)],
                sem,
            ).wait()

        def pool(h):
            def body(i, c2):
                acc = g_v[0, pl.ds(i * 16, 16)]
                for s in range(1, S):
                    acc = acc + g_v[s, pl.ds(i * 16, 16)]
                out_v[pl.ds(i * 16, 16)] = acc * inv
                return c2

            lax.fori_loop(h * (half // 16), (h + 1) * (half // 16), body, 0)

        stage_main(d_base, 0)
        stage_tail(d_base)
        stage_main_wait()
        stage_tail_wait()
        plsc.subcore_barrier()

        for dd in range(d_per_core):
            p = dd % 2
            d = d_base + dd

            if dd + 1 < d_per_core:
                stage_main(d + 1, 1 - p)

            fire(p, 0)
            drain(0)
            fire(p, 1)
            drain(1)

            plsc.subcore_barrier()
            if dd + 1 < d_per_core:
                stage_tail(d + 1)

            pool(0)
            pool(1)
            pltpu.sync_copy(out_v, out_hbm.at[d, pl.ds(b0, b_per_tile)])

            if dd + 1 < d_per_core:
                stage_main_wait()
                stage_tail_wait()
                plsc.subcore_barrier()

    return run


def kernel(bin_indices, table):
    B, S = bin_indices.shape
    V, D = table.shape
    run = _make_sc_kernel(B, S, D, V)
    aux = table[V - 128:, :].T
    out_t = run(table.T, bin_indices.T.astype(jnp.int32), aux)
    return out_t.T


# speedup vs baseline: 1.0066x; 1.0066x over previous
import functools

import jax
import jax.numpy as jnp
from jax import lax
from jax.experimental import pallas as pl
from jax.experimental.pallas import tpu as pltpu
from jax.experimental.pallas import tpu_sc as plsc

NC = 2
NS = 16
CH = 128
M = 654592


def _make_sc_kernel(B, S, D, V):
    d_per_core = D // NC
    b_per_tile = B // NS
    half = b_per_tile // 2
    n_ch_half = (S * half) // CH
    kh = half // CH
    t_main = (V - V % 128) - M
    t_size = V - M
    arena = 2 * M + t_size + 64
    seg = (M // NS) // 128 * 128
    seg_last = M - seg * (NS - 1)
    tseg = (t_main // NS) // 128 * 128
    tseg_last = t_main - tseg * (NS - 1)
    mesh = plsc.VectorSubcoreMesh(core_axis_name="c", subcore_axis_name="s")

    @functools.partial(
        pl.kernel,
        out_type=jax.ShapeDtypeStruct((D, B), jnp.float32),
        mesh=mesh,
        scratch_types=[
            pltpu.VMEM((S, b_per_tile), jnp.int32),
            pltpu.VMEM((S, b_per_tile), jnp.int32),
            pltpu.VMEM((S, b_per_tile), jnp.float32),
            pltpu.VMEM((b_per_tile,), jnp.float32),
            pltpu.VMEM((D, 128), jnp.float32),
            pltpu.VMEM_SHARED((arena,), jnp.float32),
            pltpu.SemaphoreType.DMA,
            pltpu.SemaphoreType.DMA,
        ],
    )
    def run(tbl_hbm, idx_hbm, aux_hbm, out_hbm, raw_v, ia_v, g_v,
            out_v, aux_v, arena_sh, sem, sem_stage):
        cid = lax.axis_index("c")
        sid = lax.axis_index("s")
        b0 = sid * b_per_tile
        d_base = cid * d_per_core
        c0 = pl.multiple_of(sid * seg, 128)
        tc0 = pl.multiple_of(sid * tseg, 128)

        pltpu.sync_copy(idx_hbm.at[:, pl.ds(b0, b_per_tile)], raw_v)

        @pl.when(sid == NS - 1)
        def _stage_aux():
            pltpu.sync_copy(aux_hbm, aux_v)

        m_c = jnp.int32(M)

        def xform(j, c2):
            s = j // (b_per_tile // 16)
            i = j % (b_per_tile // 16)
            x = raw_v[s, pl.ds(i * 16, 16)]
            ia_v[s, pl.ds(i * 16, 16)] = jnp.where(x >= m_c, x + m_c, x)
            return c2

        lax.fori_loop(0, S * (b_per_tile // 16), xform, 0)

        inv = jnp.float32(1.0 / S)
        ref_b = arena_sh.at[pl.ds(M, V + 64)]

        def stage_main(d, p):
            base = p * M

            @pl.when(sid < NS - 1)
            def _seg():
                pltpu.async_copy(
                    tbl_hbm.at[d].at[pl.ds(c0, seg)],
                    arena_sh.at[pl.ds(base + c0, seg)],
                    sem_stage,
                )

            @pl.when(sid == NS - 1)
            def _seg_last():
                pltpu.async_copy(
                    tbl_hbm.at[d].at[pl.ds(c0, seg_last)],
                    arena_sh.at[pl.ds(base + c0, seg_last)],
                    sem_stage,
                )

        def stage_main_wait():
            @pl.when(sid < NS - 1)
            def _seg():
                pltpu.make_async_copy(
                    tbl_hbm.at[0].at[pl.ds(0, seg)],
                    arena_sh.at[pl.ds(0, seg)],
                    sem_stage,
                ).wait()

            @pl.when(sid == NS - 1)
            def _seg_last():
                pltpu.make_async_copy(
                    tbl_hbm.at[0].at[pl.ds(0, seg_last)],
                    arena_sh.at[pl.ds(0, seg_last)],
                    sem_stage,
                ).wait()

        def stage_tail(d):
            @pl.when(sid < NS - 1)
            def _t():
                pltpu.async_copy(
                    tbl_hbm.at[d].at[pl.ds(M + tc0, tseg)],
                    arena_sh.at[pl.ds(2 * M + tc0, tseg)],
                    sem_stage,
                )

            @pl.when(sid == NS - 1)
            def _t_last():
                pltpu.async_copy(
                    tbl_hbm.at[d].at[pl.ds(M + tc0, tseg_last)],
                    arena_sh.at[pl.ds(2 * M + tc0, tseg_last)],
                    sem_stage,
                )
                pltpu.async_copy(
                    aux_v.at[d],
                    arena_sh.at[pl.ds(2 * M + t_size - 128, 128)],
                    sem_stage,
                )

        def stage_tail_wait():
            @pl.when(sid < NS - 1)
            def _t():
                pltpu.make_async_copy(
                    tbl_hbm.at[0].at[pl.ds(0, tseg)],
                    arena_sh.at[pl.ds(0, tseg)],
                    sem_stage,
                ).wait()

            @pl.when(sid == NS - 1)
            def _t_last():
                pltpu.make_async_copy(
                    tbl_hbm.at[0].at[pl.ds(0, tseg_last)],
                    arena_sh.at[pl.ds(0, tseg_last)],
                    sem_stage,
                ).wait()
                pltpu.make_async_copy(
                    aux_v.at[0],
                    arena_sh.at[pl.ds(0, 128)],
                    sem_stage,
                ).wait()

        def fire(p, h):
            def body(k, c2):
                off = h * half + k * CH
                for s in range(S):
                    if p == 0:
                        src = arena_sh.at[ia_v.at[s, pl.ds(off, CH)]]
                    else:
                        src = ref_b.at[raw_v.at[s, pl.ds(off, CH)]]
                    pltpu.async_copy(src, g_v.at[s, pl.ds(off, CH)], sem)
                return c2

            lax.fori_loop(0, kh, body, 0)

        def drain(h):
            pltpu.make_async_copy(
                tbl_hbm.at[pl.ds(0, S), pl.ds(0, half)],
                g_v.at[:, pl.ds(h * half, half)],
                sem,
            ).wait()

        def pool(h):
            def body(i, c2):
                acc = g_v[0, pl.ds(i * 16, 16)]
                for s in range(1, S):
                    acc = acc + g_v[s, pl.ds(i * 16, 16)]
                out_v[pl.ds(i * 16, 16)] = acc * inv
                return c2

            lax.fori_loop(h * (half // 16), (h + 1) * (half // 16), body, 0)

        stage_main(d_base, 0)
        stage_tail(d_base)
        stage_main_wait()
        stage_tail_wait()
        plsc.subcore_barrier()

        for dd in range(d_per_core):
            p = dd % 2
            d = d_base + dd

            if dd + 1 < d_per_core:
                stage_main(d + 1, 1 - p)

            fire(p, 0)
            drain(0)
            fire(p, 1)
            pool(0)
            drain(1)

            plsc.subcore_barrier()
            if dd + 1 < d_per_core:
                stage_tail(d + 1)

            pool(1)
            pltpu.sync_copy(out_v, out_hbm.at[d, pl.ds(b0, b_per_tile)])

            if dd + 1 < d_per_core:
                stage_main_wait()
                stage_tail_wait()
                plsc.subcore_barrier()

    return run


def kernel(bin_indices, table):
    B, S = bin_indices.shape
    V, D = table.shape
    run = _make_sc_kernel(B, S, D, V)
    aux = table[V - 128:, :].T
    out_t = run(table.T, bin_indices.T.astype(jnp.int32), aux)
    return out_t.T
